# Initial kernel scaffold; baseline (speedup 1.0000x reference)
import jax
import jax.numpy as jnp
from jax.experimental import pallas as pl


def kernel(exercise_ids, desc_tokens, exercise_table, desc_table):
    raise NotImplementedError("write your pallas kernel here")


# speedup vs baseline: 23.4815x; 23.4815x over previous
import functools

import jax
import jax.numpy as jnp
from jax import lax
from jax.experimental import pallas as pl
from jax.experimental.pallas import tpu as pltpu
from jax.experimental.pallas import tpu_sc as plsc

_BATCH = 16384
_LEN = 50
_LEN_PAD = 64
_EMB = 32
_DESC_ROWS = 1500
_NC = 2
_NS = 16
_NW = _NC * _NS
_BPW = _BATCH // _NW
_NG = _BPW // 16

_mesh = plsc.VectorSubcoreMesh(core_axis_name="c", subcore_axis_name="s")


@functools.partial(
    pl.kernel,
    out_type=jax.ShapeDtypeStruct((_BATCH * 2 * _EMB,), jnp.float32),
    mesh=_mesh,
    scratch_types=[
        pltpu.VMEM((_DESC_ROWS * _EMB,), jnp.float32),
        pltpu.VMEM((_BPW * _LEN_PAD,), jnp.int32),
        pltpu.VMEM((_BPW,), jnp.int32),
        pltpu.VMEM((_BPW * 2 * _EMB,), jnp.float32),
        pltpu.SemaphoreType.DMA,
        pltpu.SemaphoreType.DMA,
        pltpu.SemaphoreType.DMA,
    ],
)
def _sc_embed(ids_hbm, toks_hbm, extab_hbm, dtab_hbm, out_hbm,
              dtab_v, toks_v, ids_v, out_v,
              sem_tab, sem_tok, sem_ex):
    wid = lax.axis_index("s") * _NC + lax.axis_index("c")
    base = wid * _BPW

    pltpu.sync_copy(ids_hbm.at[pl.ds(base, _BPW)], ids_v)
    cp_tok = pltpu.async_copy(
        toks_hbm.at[pl.ds(base * _LEN_PAD, _BPW * _LEN_PAD)], toks_v, sem_tok)
    cp_tab = pltpu.async_copy(dtab_hbm, dtab_v, sem_tab)

    def fire(g, _):
        idv = ids_v[pl.ds(g * 16, 16)]
        for k in range(16):
            rid = idv[k]
            pltpu.async_copy(
                extab_hbm.at[pl.ds(rid * _EMB, _EMB)],
                out_v.at[pl.ds((g * 16 + k) * 2 * _EMB, _EMB)], sem_ex)
        return 0
    lax.fori_loop(0, _NG, fire, 0)

    cp_tab.wait()
    z16 = jnp.zeros((16,), jnp.float32)
    dtab_v[pl.ds(0, 16)] = z16
    dtab_v[pl.ds(16, 16)] = z16
    cp_tok.wait()

    def row(i, _):
        tb = i * _LEN_PAD
        a0 = z16
        a1 = z16
        cnt = jnp.int32(0)
        for c in range(4):
            chunk = toks_v[pl.ds(tb + c * 16, 16)]
            for k in range(16 if c < 3 else _LEN - 48):
                tok = chunk[k]
                rb = tok * _EMB
                cnt = cnt + jnp.where(tok != 0, 1, 0)
                a0 = a0 + dtab_v[pl.ds(rb, 16)]
                a1 = a1 + dtab_v[pl.ds(rb + 16, 16)]
        cntv = jnp.full((16,), cnt, jnp.float32)
        rcp = 1.0 / jnp.maximum(cntv, 1.0)
        ob = i * 2 * _EMB
        out_v[pl.ds(ob + 32, 16)] = a0 * rcp
        out_v[pl.ds(ob + 48, 16)] = a1 * rcp
        return 0
    lax.fori_loop(0, _BPW, row, 0)

    pltpu.make_async_copy(
        extab_hbm.at[pl.ds(0, _BPW * _EMB)],
        out_v.at[pl.ds(0, _BPW * _EMB)], sem_ex).wait()
    pltpu.sync_copy(out_v, out_hbm.at[pl.ds(base * 2 * _EMB, _BPW * 2 * _EMB)])


def kernel(exercise_ids, desc_tokens, exercise_table, desc_table):
    ids = exercise_ids.astype(jnp.int32).reshape(-1)
    toks = jnp.pad(desc_tokens.astype(jnp.int32),
                   ((0, 0), (0, _LEN_PAD - _LEN))).reshape(-1)
    out = _sc_embed(ids, toks, exercise_table.reshape(-1),
                    desc_table.reshape(-1))
    return out.reshape(_BATCH, 2 * _EMB)


# speedup vs baseline: 24.3637x; 1.0376x over previous
import functools

import jax
import jax.numpy as jnp
from jax import lax
from jax.experimental import pallas as pl
from jax.experimental.pallas import tpu as pltpu
from jax.experimental.pallas import tpu_sc as plsc

_BATCH = 16384
_LEN = 50
_LEN_PAD = 64
_EMB = 32
_DESC_ROWS = 1500
_NC = 2
_NS = 16
_NW = _NC * _NS
_BPW = _BATCH // _NW
_NG = _BPW // 16

_mesh = plsc.VectorSubcoreMesh(core_axis_name="c", subcore_axis_name="s")


@functools.partial(
    pl.kernel,
    out_type=jax.ShapeDtypeStruct((_BATCH * 2 * _EMB,), jnp.float32),
    mesh=_mesh,
    scratch_types=[
        pltpu.VMEM((_DESC_ROWS * _EMB,), jnp.float32),
        pltpu.VMEM((_BPW * _LEN_PAD,), jnp.int32),
        pltpu.VMEM((_BPW,), jnp.int32),
        pltpu.VMEM((_BPW * 2 * _EMB,), jnp.float32),
        pltpu.SemaphoreType.DMA,
        pltpu.SemaphoreType.DMA,
        pltpu.SemaphoreType.DMA,
    ],
)
def _sc_embed(ids_hbm, toks_hbm, extab_hbm, dtab_hbm, out_hbm,
              dtab_v, toks_v, ids_v, out_v,
              sem_tab, sem_tok, sem_ex):
    wid = lax.axis_index("s") * _NC + lax.axis_index("c")
    base = wid * _BPW

    pltpu.sync_copy(ids_hbm.at[pl.ds(base, _BPW)], ids_v)
    cp_tok = pltpu.async_copy(
        toks_hbm.at[pl.ds(base * _LEN_PAD, _BPW * _LEN_PAD)], toks_v, sem_tok)
    cp_tab = pltpu.async_copy(dtab_hbm, dtab_v, sem_tab)

    @plsc.parallel_loop(0, _NG)
    def fire(g):
        idv = ids_v[pl.ds(g * 16, 16)]
        for k in range(16):
            rid = idv[k]
            pltpu.async_copy(
                extab_hbm.at[pl.ds(rid * _EMB, _EMB)],
                out_v.at[pl.ds((g * 16 + k) * 2 * _EMB, _EMB)], sem_ex)

    cp_tab.wait()
    z16 = jnp.zeros((16,), jnp.float32)
    dtab_v[pl.ds(0, 16)] = z16
    dtab_v[pl.ds(16, 16)] = z16
    cp_tok.wait()

    @plsc.parallel_loop(0, _BPW, unroll=2)
    def row(i):
        tb = i * _LEN_PAD
        a0 = z16
        a1 = z16
        cnt = jnp.int32(0)
        for c in range(4):
            chunk = toks_v[pl.ds(tb + c * 16, 16)]
            for k in range(16 if c < 3 else _LEN - 48):
                off = chunk[k]
                cnt = cnt + jnp.minimum(off, 1)
                a0 = a0 + dtab_v[pl.ds(off, 16)]
                a1 = a1 + dtab_v[pl.ds(off + 16, 16)]
        cntv = jnp.full((16,), cnt, jnp.float32)
        rcp = 1.0 / jnp.maximum(cntv, 1.0)
        ob = i * 2 * _EMB
        out_v[pl.ds(ob + 32, 16)] = a0 * rcp
        out_v[pl.ds(ob + 48, 16)] = a1 * rcp

    pltpu.make_async_copy(
        extab_hbm.at[pl.ds(0, _BPW * _EMB)],
        out_v.at[pl.ds(0, _BPW * _EMB)], sem_ex).wait()
    pltpu.sync_copy(out_v, out_hbm.at[pl.ds(base * 2 * _EMB, _BPW * 2 * _EMB)])


def kernel(exercise_ids, desc_tokens, exercise_table, desc_table):
    ids = exercise_ids.astype(jnp.int32).reshape(-1)
    toks = jnp.pad(desc_tokens.astype(jnp.int32) * _EMB,
                   ((0, 0), (0, _LEN_PAD - _LEN))).reshape(-1)
    out = _sc_embed(ids, toks, exercise_table.reshape(-1),
                    desc_table.reshape(-1))
    return out.reshape(_BATCH, 2 * _EMB)


# speedup vs baseline: 24.7208x; 1.0147x over previous
import functools

import jax
import jax.numpy as jnp
from jax import lax
from jax.experimental import pallas as pl
from jax.experimental.pallas import tpu as pltpu
from jax.experimental.pallas import tpu_sc as plsc

_BATCH = 16384
_LEN = 50
_LEN_PAD = 64
_EMB = 32
_DESC_ROWS = 1500
_NC = 2
_NS = 16
_NW = _NC * _NS
_BPW = _BATCH // _NW
_NG = _BPW // 16

_mesh = plsc.VectorSubcoreMesh(core_axis_name="c", subcore_axis_name="s")


@functools.partial(
    pl.kernel,
    out_type=jax.ShapeDtypeStruct((_BATCH * 2 * _EMB,), jnp.float32),
    mesh=_mesh,
    scratch_types=[
        pltpu.VMEM((_DESC_ROWS * _EMB,), jnp.float32),
        pltpu.VMEM((_BPW * _LEN_PAD,), jnp.int32),
        pltpu.VMEM((_BPW,), jnp.int32),
        pltpu.VMEM((_BPW * 2 * _EMB,), jnp.float32),
        pltpu.SemaphoreType.DMA,
        pltpu.SemaphoreType.DMA,
        pltpu.SemaphoreType.DMA,
    ],
)
def _sc_embed(ids_hbm, toks_hbm, extab_hbm, dtab_hbm, out_hbm,
              dtab_v, toks_v, ids_v, out_v,
              sem_tab, sem_tok, sem_ex):
    wid = lax.axis_index("s") * _NC + lax.axis_index("c")
    base = wid * _BPW

    pltpu.sync_copy(ids_hbm.at[pl.ds(base, _BPW)], ids_v)
    cp_tok = pltpu.async_copy(
        toks_hbm.at[pl.ds(base * _LEN_PAD, _BPW * _LEN_PAD)], toks_v, sem_tok)
    cp_tab = pltpu.async_copy(dtab_hbm, dtab_v, sem_tab)

    @plsc.parallel_loop(0, _NG)
    def fire(g):
        idv = ids_v[pl.ds(g * 16, 16)]
        for k in range(16):
            rid = idv[k]
            pltpu.async_copy(
                extab_hbm.at[pl.ds(rid * _EMB, _EMB)],
                out_v.at[pl.ds((g * 16 + k) * 2 * _EMB, _EMB)], sem_ex)

    cp_tab.wait()
    z16 = jnp.zeros((16,), jnp.float32)
    dtab_v[pl.ds(0, 16)] = z16
    dtab_v[pl.ds(16, 16)] = z16
    cp_tok.wait()

    @plsc.parallel_loop(0, _BPW, unroll=2)
    def row(i):
        tb = i * _LEN_PAD
        acc = [z16, z16, z16, z16]
        cnt = jnp.int32(0)
        for c in range(4):
            chunk = toks_v[pl.ds(tb + c * 16, 16)]
            p = 2 * (c // 2)
            for k in range(16 if c < 3 else _LEN - 48):
                off = chunk[k]
                cnt = cnt + jnp.minimum(off, 1)
                acc[p] = acc[p] + dtab_v[pl.ds(off, 16)]
                acc[p + 1] = acc[p + 1] + dtab_v[pl.ds(off + 16, 16)]
        cntv = jnp.full((16,), cnt, jnp.float32)
        rcp = 1.0 / jnp.maximum(cntv, 1.0)
        ob = i * 2 * _EMB
        out_v[pl.ds(ob + 32, 16)] = (acc[0] + acc[2]) * rcp
        out_v[pl.ds(ob + 48, 16)] = (acc[1] + acc[3]) * rcp

    pltpu.make_async_copy(
        extab_hbm.at[pl.ds(0, _BPW * _EMB)],
        out_v.at[pl.ds(0, _BPW * _EMB)], sem_ex).wait()
    pltpu.sync_copy(out_v, out_hbm.at[pl.ds(base * 2 * _EMB, _BPW * 2 * _EMB)])


def kernel(exercise_ids, desc_tokens, exercise_table, desc_table):
    ids = exercise_ids.astype(jnp.int32).reshape(-1)
    toks = jnp.pad(desc_tokens.astype(jnp.int32) * _EMB,
                   ((0, 0), (0, _LEN_PAD - _LEN))).reshape(-1)
    out = _sc_embed(ids, toks, exercise_table.reshape(-1),
                    desc_table.reshape(-1))
    return out.reshape(_BATCH, 2 * _EMB)
